# baseline (speedup 1.0000x reference)
import functools

import jax
import jax.numpy as jnp
from jax import lax
from jax.experimental import pallas as pl
from jax.experimental.pallas import tpu as pltpu
from jax.experimental.pallas import tpu_sc as plsc

SITE_EMBED_DIM = 64

_NC = 2
_NS = 16
_NW = _NC * _NS

_C = 128
_G = 4


def _gather_kernel(
    n_super, idx_hbm, table_hbm, out_hbm, idx_v, table_v, rows_v, g0, g1, w0, w1
):
    wid = lax.axis_index("s") * _NC + lax.axis_index("c")
    base = wid * (n_super * _G * _C)
    n_chunks = n_super * _G

    sid = lax.axis_index("s")
    @pl.when(sid == 0)
    def _():
        pltpu.sync_copy(table_hbm, table_v)
    pltpu.sync_copy(idx_hbm.at[wid], idx_v)
    plsc.subcore_barrier()

    def gather(t, slot, gsem):
        pltpu.async_copy(
            table_v.at[idx_v.at[pl.ds(t * _G * _C, _G * _C)]], rows_v.at[slot], gsem
        )

    def gather_wait(slot, gsem):
        pltpu.make_async_copy(
            table_v.at[idx_v.at[pl.ds(0, _G * _C)]], rows_v.at[slot], gsem
        ).wait()

    def write(t, slot, wsem):
        pltpu.async_copy(
            rows_v.at[slot],
            out_hbm.at[pl.ds(base + t * _G * _C, _G * _C), pl.ds(0, 64)],
            wsem,
        )

    def write_wait(slot, wsem):
        pltpu.make_async_copy(
            rows_v.at[slot], out_hbm.at[pl.ds(base, _G * _C), pl.ds(0, 64)], wsem
        ).wait()

    gather(0, 0, g0)
    gather(1, 1, g1)

    def body(tt, carry):
        t0 = 2 * tt
        t1 = t0 + 1
        gather_wait(0, g0)
        write(t0, 0, w0)
        write_wait(0, w0)
        gather(t0 + 2, 0, g0)
        gather_wait(1, g1)
        write(t1, 1, w1)
        write_wait(1, w1)
        gather(t1 + 2, 1, g1)
        return carry

    lax.fori_loop(0, n_super // 2 - 1, body, 0)

    t0 = n_super - 2
    t1 = n_super - 1
    gather_wait(0, g0)
    write(t0, 0, w0)
    gather_wait(1, g1)
    write(t1, 1, w1)
    write_wait(0, w0)
    write_wait(1, w1)


def kernel(x, embed_site):
    n_rows, n_cols = x.shape
    d = embed_site.shape[1]
    total = n_rows * n_cols
    assert total % (_NW * _C * _G) == 0
    n_super = total // (_NW * _C * _G)
    n_chunks = n_super * _G

    idx = x.reshape(_NW, n_chunks * _C).astype(jnp.int32)

    mesh = plsc.VectorSubcoreMesh(
        core_axis_name="c", subcore_axis_name="s", num_cores=_NC, num_subcores=_NS
    )
    run = pl.kernel(
        functools.partial(_gather_kernel, n_super),
        out_type=jax.ShapeDtypeStruct((total, 2 * d), jnp.float32),
        mesh=mesh,
        scratch_types=[
            pltpu.VMEM((n_chunks * _C,), jnp.int32),
            pltpu.VMEM_SHARED(embed_site.shape, jnp.float32),
            pltpu.VMEM((2, _G * _C, d), jnp.float32),
            pltpu.SemaphoreType.DMA,
            pltpu.SemaphoreType.DMA,
            pltpu.SemaphoreType.DMA,
            pltpu.SemaphoreType.DMA,
        ],
        compiler_params=pltpu.CompilerParams(use_tc_tiling_on_sc=False),
    )
    padded = run(idx, embed_site)
    return _tc_slice(padded, d)


def _tc_slice_body(x_ref, o_ref):
    o_ref[...] = x_ref[:, : o_ref.shape[1]]


def _tc_slice(padded, d):
    total = padded.shape[0]
    blk = 4096
    return pl.pallas_call(
        _tc_slice_body,
        grid=(total // blk,),
        in_specs=[pl.BlockSpec((blk, padded.shape[1]), lambda i: (i, 0))],
        out_specs=pl.BlockSpec((blk, d), lambda i: (i, 0)),
        out_shape=jax.ShapeDtypeStruct((total, d), jnp.float32),
    )(padded)


# speedup vs baseline: 1.5158x; 1.5158x over previous
import functools

import jax
import jax.numpy as jnp
from jax import lax
from jax.experimental import pallas as pl
from jax.experimental.pallas import tpu as pltpu
from jax.experimental.pallas import tpu_sc as plsc

SITE_EMBED_DIM = 64

_NC = 2
_NS = 16
_NW = _NC * _NS

_C = 128


def _gather_kernel(
    n_chunks, idx_hbm, table_hbm, out_hbm, idx_v, table_v, rows_v, g0, g1, w0, w1
):
    wid = lax.axis_index("s") * _NC + lax.axis_index("c")
    base = wid * (n_chunks * _C)

    sid = lax.axis_index("s")

    @pl.when(sid == 0)
    def _():
        pltpu.sync_copy(table_hbm, table_v)

    pltpu.sync_copy(idx_hbm.at[wid], idx_v)
    plsc.subcore_barrier()

    def gather(t, slot, gsem):
        pltpu.async_copy(table_v.at[idx_v.at[t]], rows_v.at[slot], gsem)

    def gather_wait(slot, gsem):
        pltpu.make_async_copy(
            table_v.at[idx_v.at[0]], rows_v.at[slot], gsem
        ).wait()

    def write(t, slot, wsem):
        pltpu.async_copy(
            rows_v.at[slot], out_hbm.at[pl.ds(base + t * _C, _C)], wsem
        )

    def write_wait(slot, wsem):
        pltpu.make_async_copy(
            rows_v.at[slot], out_hbm.at[pl.ds(base, _C)], wsem
        ).wait()

    gather(0, 0, g0)
    gather(1, 1, g1)

    def body(tt, carry):
        t0 = 2 * tt
        t1 = t0 + 1
        gather_wait(0, g0)
        write(t0, 0, w0)
        write_wait(0, w0)
        gather(t0 + 2, 0, g0)
        gather_wait(1, g1)
        write(t1, 1, w1)
        write_wait(1, w1)
        gather(t1 + 2, 1, g1)
        return carry

    lax.fori_loop(0, n_chunks // 2 - 1, body, 0)

    t0 = n_chunks - 2
    t1 = n_chunks - 1
    gather_wait(0, g0)
    write(t0, 0, w0)
    gather_wait(1, g1)
    write(t1, 1, w1)
    write_wait(0, w0)
    write_wait(1, w1)


def kernel(x, embed_site):
    n_rows, n_cols = x.shape
    d = embed_site.shape[1]
    total = n_rows * n_cols
    assert total % (_NW * _C) == 0
    n_chunks = total // (_NW * _C)

    idx = x.reshape(_NW, n_chunks, _C).astype(jnp.int32)

    mesh = plsc.VectorSubcoreMesh(
        core_axis_name="c", subcore_axis_name="s", num_cores=_NC, num_subcores=_NS
    )
    run = pl.kernel(
        functools.partial(_gather_kernel, n_chunks),
        out_type=jax.ShapeDtypeStruct((total, d), jnp.float32),
        mesh=mesh,
        scratch_types=[
            pltpu.VMEM((n_chunks, _C), jnp.int32),
            pltpu.VMEM_SHARED(embed_site.shape, jnp.float32),
            pltpu.VMEM((2, _C, d), jnp.float32),
            pltpu.SemaphoreType.DMA,
            pltpu.SemaphoreType.DMA,
            pltpu.SemaphoreType.DMA,
            pltpu.SemaphoreType.DMA,
        ],
    )
    return run(idx, embed_site)


# speedup vs baseline: 2.3157x; 1.5277x over previous
import functools

import jax
import jax.numpy as jnp
from jax import lax
from jax.experimental import pallas as pl
from jax.experimental.pallas import tpu as pltpu
from jax.experimental.pallas import tpu_sc as plsc

SITE_EMBED_DIM = 64

_NC = 2
_NS = 16
_NW = _NC * _NS

_C = 128
_G = 4


def _gather_kernel(
    n_super, idx_hbm, table_hbm, out_hbm, idx_v, table_v, rows_v, g0, g1, w0, w1
):
    wid = lax.axis_index("s") * _NC + lax.axis_index("c")
    base = wid * (n_super * _G * _C)
    n_chunks = n_super * _G

    sid = lax.axis_index("s")
    @pl.when(sid == 0)
    def _():
        pltpu.sync_copy(table_hbm, table_v)
    pltpu.sync_copy(idx_hbm.at[wid], idx_v)
    plsc.subcore_barrier()

    def gather(t, slot, gsem):
        pltpu.async_copy(
            table_v.at[idx_v.at[pl.ds(t * _G * _C, _G * _C)]], rows_v.at[slot], gsem
        )

    def gather_wait(slot, gsem):
        pltpu.make_async_copy(
            table_v.at[idx_v.at[pl.ds(0, _G * _C)]], rows_v.at[slot], gsem
        ).wait()

    def write(t, slot, wsem):
        pltpu.async_copy(
            rows_v.at[slot],
            out_hbm.at[pl.ds(base + t * _G * _C, _G * _C), pl.ds(0, 64)],
            wsem,
        )

    def write_wait(slot, wsem):
        pltpu.make_async_copy(
            rows_v.at[slot], out_hbm.at[pl.ds(base, _G * _C), pl.ds(0, 64)], wsem
        ).wait()

    gather(0, 0, g0)
    gather(1, 1, g1)

    def body(tt, carry):
        t0 = 2 * tt
        t1 = t0 + 1
        gather_wait(0, g0)
        write(t0, 0, w0)
        write_wait(0, w0)
        gather(t0 + 2, 0, g0)
        gather_wait(1, g1)
        write(t1, 1, w1)
        write_wait(1, w1)
        gather(t1 + 2, 1, g1)
        return carry

    lax.fori_loop(0, n_super // 2 - 1, body, 0)

    t0 = n_super - 2
    t1 = n_super - 1
    gather_wait(0, g0)
    write(t0, 0, w0)
    gather_wait(1, g1)
    write(t1, 1, w1)
    write_wait(0, w0)
    write_wait(1, w1)


def kernel(x, embed_site):
    n_rows, n_cols = x.shape
    d = embed_site.shape[1]
    total = n_rows * n_cols
    assert total % (_NW * _C * _G) == 0
    n_super = total // (_NW * _C * _G)
    n_chunks = n_super * _G

    idx = x.reshape(_NW, n_chunks * _C).astype(jnp.int32)

    mesh = plsc.VectorSubcoreMesh(
        core_axis_name="c", subcore_axis_name="s", num_cores=_NC, num_subcores=_NS
    )
    run = pl.kernel(
        functools.partial(_gather_kernel, n_super),
        out_type=jax.ShapeDtypeStruct((total, 2 * d), jnp.float32),
        mesh=mesh,
        scratch_types=[
            pltpu.VMEM((n_chunks * _C,), jnp.int32),
            pltpu.VMEM_SHARED(embed_site.shape, jnp.float32),
            pltpu.VMEM((2, _G * _C, d), jnp.float32),
            pltpu.SemaphoreType.DMA,
            pltpu.SemaphoreType.DMA,
            pltpu.SemaphoreType.DMA,
            pltpu.SemaphoreType.DMA,
        ],
        compiler_params=pltpu.CompilerParams(use_tc_tiling_on_sc=False),
    )
    return run(idx, embed_site)[:, :d]
